# Initial kernel scaffold; baseline (speedup 1.0000x reference)
import jax
import jax.numpy as jnp
from jax.experimental import pallas as pl


def kernel(x, gate_w, fc1_w, fc1_b, fc2_w, fc2_b):
    raise NotImplementedError("write your pallas kernel here")


# speedup vs baseline: 11.0302x; 11.0302x over previous
import functools
import math

import jax
import jax.numpy as jnp
from jax.experimental import pallas as pl
from jax.experimental.pallas import tpu as pltpu

HID = 768
FF = 3072
E = 64
N_TOK = 2048
BLK = 128
NBLK = 80
NCAP = NBLK * BLK
_SQRT2 = math.sqrt(2.0)


def _cumsum_ax0(a):
    n = a.shape[0]
    k = 1
    while k < n:
        z = jnp.zeros((k, a.shape[1]), a.dtype)
        a = a + jnp.concatenate([z, a[: n - k]], axis=0)
        k *= 2
    return a


def _cumsum_ax1(a):
    n = a.shape[1]
    k = 1
    while k < n:
        z = jnp.zeros((a.shape[0], k), a.dtype)
        a = a + jnp.concatenate([z, a[:, : n - k]], axis=1)
        k *= 2
    return a


def _router_kernel(x_ref, gw_ref, noise_ref,
                   logits_ref, gating_ref, lp_ref, topi_ref, w_ref,
                   rid_ref, be_ref, nu_ref):
    x = x_ref[:]
    gw = gw_ref[:]
    logits = jax.lax.dot_general(
        x, gw, (((1,), (1,)), ((), ())), preferred_element_type=jnp.float32)
    logits_ref[:] = logits
    m = jnp.max(logits, axis=1, keepdims=True)
    ex = jnp.exp(logits - m)
    gating = ex / jnp.sum(ex, axis=1, keepdims=True)
    gating_ref[:] = gating
    noisy = logits + noise_ref[:]
    tau = jnp.max(noisy, axis=1, keepdims=True)
    z = (tau - logits) * float(E)
    lp_ref[:] = 0.5 * (1.0 - jax.lax.erf(z * (1.0 / _SQRT2)))
    iota_e = jax.lax.broadcasted_iota(jnp.int32, (N_TOK, E), 1)
    topi = jnp.min(jnp.where(logits == m, iota_e, E), axis=1, keepdims=True)
    topi_ref[:] = topi
    topw = jnp.max(gating, axis=1, keepdims=True)
    w_ref[:] = topw / (topw + 1e-9)
    mi = (topi == iota_e).astype(jnp.int32)
    counts = jnp.sum(mi, axis=0, keepdims=True)
    pc = ((counts + (BLK - 1)) // BLK) * BLK
    ends = _cumsum_ax1(pc)
    offs = ends - pc
    csum = _cumsum_ax0(mi)
    rank = jnp.sum(csum * mi, axis=1, keepdims=True) - 1
    rid_ref[:] = jnp.sum(mi * offs, axis=1, keepdims=True) + rank
    bstart = jax.lax.broadcasted_iota(jnp.int32, (NBLK, E), 0) * BLK
    ends_b = jnp.broadcast_to(ends, (NBLK, E))
    be = jnp.sum((ends_b <= bstart).astype(jnp.int32), axis=1, keepdims=True)
    be_ref[:] = jnp.minimum(be, E - 1)
    nu_ref[:] = ends[:, E - 1:E]


def _gather_kernel(rid_ref, x_ref, xs_ref):
    def body(i, c):
        r = rid_ref[i]
        xs_ref[pl.ds(r, 1), :] = x_ref[pl.ds(i, 1), :]
        return c
    jax.lax.fori_loop(0, N_TOK, body, 0)


def _gmm_kernel(be_ref, nu_ref, xs_ref, w1_ref, b1_ref, w2_ref, b2_ref,
                ys_ref):
    b = pl.program_id(0)

    @pl.when(b * BLK < nu_ref[0])
    def _():
        xb = xs_ref[:]
        w1 = w1_ref[0]
        h = jax.lax.dot_general(
            xb, w1, (((1,), (1,)), ((), ())), preferred_element_type=jnp.float32)
        h = h + b1_ref[0]
        h = 0.5 * h * (1.0 + jax.lax.erf(h * (1.0 / _SQRT2)))
        w2 = w2_ref[0]
        y = jax.lax.dot_general(
            h, w2, (((1,), (1,)), ((), ())), preferred_element_type=jnp.float32)
        ys_ref[:] = y + b2_ref[0]


def _unsort_kernel(rid_ref, ys_ref, w_ref, out_ref):
    def body(i, c):
        r = rid_ref[i]
        out_ref[pl.ds(i, 1), :] = ys_ref[pl.ds(r, 1), :]
        return c
    jax.lax.fori_loop(0, N_TOK, body, 0)
    out_ref[:] = out_ref[:] * w_ref[:]


def kernel(x, gate_w, fc1_w, fc1_b, fc2_w, fc2_b):
    B, S, D = x.shape
    xf = x.reshape(S, D)
    noise = jax.random.normal(jax.random.key(42), (S, E), dtype=jnp.float32) * (1.0 / E)

    f32 = jnp.float32
    i32 = jnp.int32
    logits, gating, lp, topi, w, rid, be, nu = pl.pallas_call(
        _router_kernel,
        out_shape=[
            jax.ShapeDtypeStruct((S, E), f32),
            jax.ShapeDtypeStruct((S, E), f32),
            jax.ShapeDtypeStruct((S, E), f32),
            jax.ShapeDtypeStruct((S, 1), i32),
            jax.ShapeDtypeStruct((S, 1), f32),
            jax.ShapeDtypeStruct((S, 1), i32),
            jax.ShapeDtypeStruct((NBLK, 1), i32),
            jax.ShapeDtypeStruct((1, 1), i32),
        ],
    )(xf, gate_w, noise)

    rid1 = rid.reshape(S)
    be1 = be.reshape(NBLK)
    nu1 = nu.reshape(1)

    xs = pl.pallas_call(
        _gather_kernel,
        grid_spec=pltpu.PrefetchScalarGridSpec(
            num_scalar_prefetch=1,
            grid=(1,),
            in_specs=[pl.BlockSpec((S, D), lambda i, rid: (0, 0))],
            out_specs=pl.BlockSpec((NCAP, D), lambda i, rid: (0, 0)),
        ),
        out_shape=jax.ShapeDtypeStruct((NCAP, D), f32),
    )(rid1, xf)

    ys = pl.pallas_call(
        _gmm_kernel,
        grid_spec=pltpu.PrefetchScalarGridSpec(
            num_scalar_prefetch=2,
            grid=(NBLK,),
            in_specs=[
                pl.BlockSpec((BLK, D), lambda b, be, nu: (b, 0)),
                pl.BlockSpec((1, FF, D), lambda b, be, nu: (be[b], 0, 0)),
                pl.BlockSpec((1, 1, FF), lambda b, be, nu: (be[b], 0, 0)),
                pl.BlockSpec((1, D, FF), lambda b, be, nu: (be[b], 0, 0)),
                pl.BlockSpec((1, 1, D), lambda b, be, nu: (be[b], 0, 0)),
            ],
            out_specs=pl.BlockSpec((BLK, D), lambda b, be, nu: (b, 0)),
        ),
        out_shape=jax.ShapeDtypeStruct((NCAP, D), f32),
    )(be1, nu1, xs, fc1_w, fc1_b.reshape(E, 1, FF), fc2_w, fc2_b.reshape(E, 1, D))

    out2d = pl.pallas_call(
        _unsort_kernel,
        grid_spec=pltpu.PrefetchScalarGridSpec(
            num_scalar_prefetch=1,
            grid=(1,),
            in_specs=[
                pl.BlockSpec((NCAP, D), lambda i, rid: (0, 0)),
                pl.BlockSpec((S, 1), lambda i, rid: (0, 0)),
            ],
            out_specs=pl.BlockSpec((S, D), lambda i, rid: (0, 0)),
        ),
        out_shape=jax.ShapeDtypeStruct((S, D), f32),
    )(rid1, ys, w)

    output = out2d.reshape(B, S, D)
    return (output, gating, logits, lp, topi)
